# baseline (speedup 1.0000x reference)
import functools

import jax
import jax.numpy as jnp
from jax import lax
from jax.experimental import pallas as pl
from jax.experimental.pallas import tpu as pltpu
from jax.experimental.pallas import tpu_sc as plsc

NC = 2
NS = 16
RW = 128
CH = 128
CORE0_FRAC = 0.36


def _ceil_div(a, b):
    return -(-a // b)


def _writeout(acc, out, c, s, n_rows, tile_rows):
    full = n_rows // tile_rows
    rem = n_rows % tile_rows

    @pl.when(s < full)
    def _():
        pltpu.sync_copy(acc.at[pl.ds(s * tile_rows, tile_rows)],
                        out.at[c, pl.ds(s * tile_rows, tile_rows)])

    if rem:
        @pl.when(s == full)
        def _():
            pltpu.sync_copy(acc.at[pl.ds(full * tile_rows, rem)],
                            out.at[c, pl.ds(full * tile_rows, rem)])


def _make_sc_deg(n, nacc, nch_max, nch0, nch1):
    tile_rows = nacc // NS
    zchunks = tile_rows // CH
    mesh = plsc.VectorSubcoreMesh(core_axis_name="c", subcore_axis_name="s")

    @functools.partial(
        pl.kernel, mesh=mesh,
        out_type=jax.ShapeDtypeStruct((NC, n, RW), jnp.float32),
        scratch_types=[
            pltpu.VMEM((nch_max, CH), jnp.int32),
            pltpu.VMEM((CH, RW), jnp.float32),
            pltpu.VMEM_SHARED((nacc, RW), jnp.float32),
        ],
    )
    def k(dst_hbm, out_hbm, didx, rows, acc):
        c = lax.axis_index("c")
        s = lax.axis_index("s")
        nch_c = jnp.where(c == 0, nch0, nch1)
        z = jnp.zeros((16,), jnp.float32)

        def zero_body(i, carry):
            for q in range(RW // 16):
                rows[i, pl.ds(q * 16, 16)] = z
            return carry
        lax.fori_loop(0, CH, zero_body, 0)

        for b in range(zchunks):
            pltpu.sync_copy(rows, acc.at[pl.ds(s * tile_rows + b * CH, CH)])
        plsc.subcore_barrier()

        one = jnp.ones((16,), jnp.float32)

        def ones_body(i, carry):
            rows[i, pl.ds(0, 16)] = one
            return carry
        lax.fori_loop(0, CH, ones_body, 0)

        pltpu.sync_copy(dst_hbm.at[c, s], didx)

        def edge_body(j, carry):
            pltpu.sync_copy(rows, acc.at[didx.at[j]], add=True)
            return carry
        lax.fori_loop(0, nch_c, edge_body, 0)

        plsc.subcore_barrier()
        _writeout(acc, out_hbm, c, s, n, tile_rows)

    return k


def _make_sc_agg(n, d, nacc, nchp, nch0, nch1):
    tile_rows = nacc // NS
    zchunks = tile_rows // CH
    h = CH // 2
    mesh = plsc.VectorSubcoreMesh(core_axis_name="c", subcore_axis_name="s")

    @functools.partial(
        pl.kernel, mesh=mesh,
        out_type=jax.ShapeDtypeStruct((NC, n, d), jnp.float32),
        scratch_types=[
            pltpu.VMEM((nchp, CH), jnp.int32),
            pltpu.VMEM((1, CH), jnp.int32),
            pltpu.VMEM((1, CH), jnp.int32),
            pltpu.VMEM((1, CH), jnp.int32),
            pltpu.VMEM((CH, d), jnp.float32),
            pltpu.VMEM((CH, d), jnp.float32),
            pltpu.VMEM_SHARED((nacc, d), jnp.float32),
            pltpu.SemaphoreType.DMA,
            pltpu.SemaphoreType.DMA,
            pltpu.SemaphoreType.DMA,
            pltpu.SemaphoreType.DMA,
            pltpu.SemaphoreType.DMA,
            pltpu.SemaphoreType.DMA,
        ],
    )
    def k(table_hbm, pidx_hbm, out_hbm, pidx, su, du0, du1, rows0, rows1,
          acc, g0a, g0b, g1a, g1b, s0, s1):
        c = lax.axis_index("c")
        s = lax.axis_index("s")
        nch_c = jnp.where(c == 0, nch0, nch1)
        z = jnp.zeros((16,), jnp.float32)

        def zero_body(i, carry):
            for q in range(d // 16):
                rows0[i, pl.ds(q * 16, 16)] = z
                rows1[i, pl.ds(q * 16, 16)] = z
            return carry
        lax.fori_loop(0, CH, zero_body, 0)

        for b in range(zchunks):
            pltpu.sync_copy(rows0, acc.at[pl.ds(s * tile_rows + b * CH, CH)])
        plsc.subcore_barrier()

        pltpu.sync_copy(pidx_hbm.at[c, s], pidx)

        def unpack(jj, du):
            for q in range(CH // 16):
                v = pidx[jj, pl.ds(q * 16, 16)]
                su[0, pl.ds(q * 16, 16)] = jnp.bitwise_and(v, 0xFFFF)
                du[0, pl.ds(q * 16, 16)] = lax.shift_right_logical(v, 16)

        def gathers(du_unused, dst_rows, sa, sb):
            pltpu.async_copy(table_hbm.at[su.at[0, pl.ds(0, h)]],
                             dst_rows.at[pl.ds(0, h)], sa)
            pltpu.async_copy(table_hbm.at[su.at[0, pl.ds(h, h)]],
                             dst_rows.at[pl.ds(h, h)], sb)

        unpack(0, du0)
        gathers(None, rows0, g0a, g0b)
        dump = jnp.full((16,), n, jnp.int32)
        for q in range(CH // 16):
            du1[0, pl.ds(q * 16, 16)] = dump
        pltpu.async_copy(rows1, acc.at[du1.at[0]], s1, add=True)

        def pair_body(pr, carry):
            j = pr * 2
            pltpu.make_async_copy(
                table_hbm.at[pl.ds(0, h)], rows0.at[pl.ds(0, h)], g0a).wait()
            pltpu.make_async_copy(
                table_hbm.at[pl.ds(0, h)], rows0.at[pl.ds(h, h)], g0b).wait()
            pltpu.async_copy(rows0, acc.at[du0.at[0]], s0, add=True)
            pltpu.make_async_copy(table_hbm.at[pl.ds(0, CH)], rows1,
                                  s1).wait()
            unpack(j + 1, du1)
            gathers(None, rows1, g1a, g1b)
            pltpu.make_async_copy(
                table_hbm.at[pl.ds(0, h)], rows1.at[pl.ds(0, h)], g1a).wait()
            pltpu.make_async_copy(
                table_hbm.at[pl.ds(0, h)], rows1.at[pl.ds(h, h)], g1b).wait()
            pltpu.async_copy(rows1, acc.at[du1.at[0]], s1, add=True)
            pltpu.make_async_copy(table_hbm.at[pl.ds(0, CH)], rows0,
                                  s0).wait()
            unpack(j + 2, du0)
            gathers(None, rows0, g0a, g0b)
            return carry
        lax.fori_loop(0, nch_c // 2, pair_body, 0)

        pltpu.make_async_copy(
            table_hbm.at[pl.ds(0, h)], rows0.at[pl.ds(0, h)], g0a).wait()
        pltpu.make_async_copy(
            table_hbm.at[pl.ds(0, h)], rows0.at[pl.ds(h, h)], g0b).wait()
        pltpu.make_async_copy(table_hbm.at[pl.ds(0, CH)], rows1, s1).wait()

        plsc.subcore_barrier()
        _writeout(acc, out_hbm, c, s, n, tile_rows)

    return k


def _t1_body(degp, feat, fs, dinv):
    deg = degp[0, :, 0:1] + degp[1, :, 0:1] + 1.0
    di = lax.rsqrt(jnp.maximum(deg, 1.0))
    dinv[...] = di
    fs[...] = feat[...] * di


def _t2_body(aggp, fs, dinv, w1, b1, w2p, out):
    di = dinv[...]
    x = (aggp[0] + aggp[1] + fs[...]) * di
    h = jnp.maximum(jnp.dot(x, w1[...], preferred_element_type=jnp.float32)
                    + b1[...], 0.0)
    out[...] = jnp.dot(h, w2p[...], preferred_element_type=jnp.float32) * di


def _t3_body(agg2p, hw2s, dinv, b2p, out):
    t = (agg2p[0] + agg2p[1] + hw2s[...]) * dinv[...] + b2p[...]
    lg = t[:, :out.shape[1]]
    m = jnp.max(lg, axis=1, keepdims=True)
    e = jnp.exp(lg - m)
    out[...] = e / jnp.sum(e, axis=1, keepdims=True)


def kernel(feat, view, W1, b1, W2, b2):
    n, f_in = feat.shape
    e = view.shape[1]
    hid = W1.shape[1]
    c_out = W2.shape[1]
    cp = 128

    nw = NC * NS
    cg = NS * CH
    nch0 = max(2, 2 * int(round(e * CORE0_FRAC / (2 * cg))))
    e0 = nch0 * cg
    e1 = e - e0
    nch1 = 2 * _ceil_div(e1, 2 * cg)
    nch_max = max(nch0, nch1)
    nchp = nch_max + 2
    nacc = _ceil_div(n + 1, NS * CH) * NS * CH

    def part(vals, pad_val):
        a = vals[:e0].reshape(NS, nch0, CH)
        a = jnp.concatenate(
            [a, jnp.full((NS, nchp - nch0, CH), pad_val, jnp.int32)], axis=1)
        b = jnp.concatenate(
            [vals[e0:], jnp.full((nch1 * cg - e1,), pad_val, jnp.int32)])
        b = b.reshape(NS, nch1, CH)
        b = jnp.concatenate(
            [b, jnp.full((NS, nchp - nch1, CH), pad_val, jnp.int32)], axis=1)
        return jnp.stack([a, b])

    src_p = part(view[0], 0)
    dst_p = part(view[1], n)
    pidx = jnp.bitwise_or(src_p, dst_p << 16)

    w2p = jnp.concatenate(
        [W2, jnp.zeros((hid, cp - c_out), jnp.float32)], axis=1)
    b1r = b1.reshape(1, hid)
    b2p = jnp.concatenate(
        [b2, jnp.zeros((cp - c_out,), jnp.float32)]).reshape(1, cp)

    degp = _make_sc_deg(n, nacc, nchp, nch0, nch1)(dst_p)

    r = 1000
    grid = (n // r,)
    fs, dinv = pl.pallas_call(
        _t1_body,
        grid=grid,
        in_specs=[pl.BlockSpec((NC, r, RW), lambda i: (0, i, 0)),
                  pl.BlockSpec((r, f_in), lambda i: (i, 0))],
        out_specs=[pl.BlockSpec((r, f_in), lambda i: (i, 0)),
                   pl.BlockSpec((r, 1), lambda i: (i, 0))],
        out_shape=[jax.ShapeDtypeStruct((n, f_in), jnp.float32),
                   jax.ShapeDtypeStruct((n, 1), jnp.float32)],
    )(degp, feat)

    aggp = _make_sc_agg(n, f_in, nacc, nchp, nch0, nch1)(fs, pidx)

    hw2s = pl.pallas_call(
        _t2_body,
        grid=grid,
        in_specs=[pl.BlockSpec((NC, r, f_in), lambda i: (0, i, 0)),
                  pl.BlockSpec((r, f_in), lambda i: (i, 0)),
                  pl.BlockSpec((r, 1), lambda i: (i, 0)),
                  pl.BlockSpec((f_in, hid), lambda i: (0, 0)),
                  pl.BlockSpec((1, hid), lambda i: (0, 0)),
                  pl.BlockSpec((hid, cp), lambda i: (0, 0))],
        out_specs=pl.BlockSpec((r, cp), lambda i: (i, 0)),
        out_shape=jax.ShapeDtypeStruct((n, cp), jnp.float32),
    )(aggp, fs, dinv, W1, b1r, w2p)

    agg2p = _make_sc_agg(n, cp, nacc, nchp, nch0, nch1)(hw2s, pidx)

    prob = pl.pallas_call(
        _t3_body,
        grid=grid,
        in_specs=[pl.BlockSpec((NC, r, cp), lambda i: (0, i, 0)),
                  pl.BlockSpec((r, cp), lambda i: (i, 0)),
                  pl.BlockSpec((r, 1), lambda i: (i, 0)),
                  pl.BlockSpec((1, cp), lambda i: (0, 0))],
        out_specs=pl.BlockSpec((r, c_out), lambda i: (i, 0)),
        out_shape=jax.ShapeDtypeStruct((n, c_out), jnp.float32),
    )(agg2p, hw2s, dinv, b2p)

    return prob


# speedup vs baseline: 1.6039x; 1.6039x over previous
import functools

import jax
import jax.numpy as jnp
from jax import lax
from jax.experimental import pallas as pl
from jax.experimental.pallas import tpu as pltpu
from jax.experimental.pallas import tpu_sc as plsc

NC = 2
NS = 16
RW = 128
CH = 128
CORE0_FRAC = 0.36


def _ceil_div(a, b):
    return -(-a // b)


def _writeout(acc, out, c, s, n_rows, tile_rows):
    full = n_rows // tile_rows
    rem = n_rows % tile_rows

    @pl.when(s < full)
    def _():
        pltpu.sync_copy(acc.at[pl.ds(s * tile_rows, tile_rows)],
                        out.at[c, pl.ds(s * tile_rows, tile_rows)])

    if rem:
        @pl.when(s == full)
        def _():
            pltpu.sync_copy(acc.at[pl.ds(full * tile_rows, rem)],
                            out.at[c, pl.ds(full * tile_rows, rem)])


def _make_sc_deg(n, nacc, nch_max, nch0, nch1):
    tile_rows = nacc // NS
    zchunks = tile_rows // CH
    mesh = plsc.VectorSubcoreMesh(core_axis_name="c", subcore_axis_name="s")

    @functools.partial(
        pl.kernel, mesh=mesh,
        out_type=jax.ShapeDtypeStruct((NC, n, RW), jnp.float32),
        scratch_types=[
            pltpu.VMEM((nch_max, CH), jnp.int32),
            pltpu.VMEM((CH, RW), jnp.float32),
            pltpu.VMEM_SHARED((nacc, RW), jnp.float32),
        ],
    )
    def k(dst_hbm, out_hbm, didx, rows, acc):
        c = lax.axis_index("c")
        s = lax.axis_index("s")
        nch_c = jnp.where(c == 0, nch0, nch1)
        z = jnp.zeros((16,), jnp.float32)

        def zero_body(i, carry):
            for q in range(RW // 16):
                rows[i, pl.ds(q * 16, 16)] = z
            return carry
        lax.fori_loop(0, CH, zero_body, 0)

        for b in range(zchunks):
            pltpu.sync_copy(rows, acc.at[pl.ds(s * tile_rows + b * CH, CH)])
        plsc.subcore_barrier()

        one = jnp.ones((16,), jnp.float32)

        def ones_body(i, carry):
            rows[i, pl.ds(0, 16)] = one
            return carry
        lax.fori_loop(0, CH, ones_body, 0)

        pltpu.sync_copy(dst_hbm.at[c, s], didx)

        def edge_body(j, carry):
            pltpu.sync_copy(rows, acc.at[didx.at[j]], add=True)
            return carry
        lax.fori_loop(0, nch_c, edge_body, 0)

        plsc.subcore_barrier()
        _writeout(acc, out_hbm, c, s, n, tile_rows)

    return k


def _make_sc_agg(n, d, nacc, nch_max, nch0, nch1):
    tile_rows = nacc // NS
    zchunks = tile_rows // CH
    mesh = plsc.VectorSubcoreMesh(core_axis_name="c", subcore_axis_name="s")

    @functools.partial(
        pl.kernel, mesh=mesh,
        out_type=jax.ShapeDtypeStruct((NC, n, d), jnp.float32),
        scratch_types=[
            pltpu.VMEM((nch_max, CH), jnp.int32),
            pltpu.VMEM((nch_max, CH), jnp.int32),
            pltpu.VMEM((CH, d), jnp.float32),
            pltpu.VMEM_SHARED((nacc, d), jnp.float32),
            pltpu.SemaphoreType.DMA,
            pltpu.SemaphoreType.DMA,
            pltpu.SemaphoreType.DMA,
            pltpu.SemaphoreType.DMA,
        ],
    )
    def k(table_hbm, src_hbm, dst_hbm, out_hbm, sidx, didx, rows, acc, sem,
          sem2, sem3, sem4):
        c = lax.axis_index("c")
        s = lax.axis_index("s")
        nch_c = jnp.where(c == 0, nch0, nch1)
        z = jnp.zeros((16,), jnp.float32)

        def zero_body(i, carry):
            for q in range(d // 16):
                rows[i, pl.ds(q * 16, 16)] = z
            return carry
        lax.fori_loop(0, CH, zero_body, 0)

        for b in range(zchunks):
            pltpu.sync_copy(rows, acc.at[pl.ds(s * tile_rows + b * CH, CH)])
        plsc.subcore_barrier()

        pltpu.sync_copy(src_hbm.at[c, s], sidx)
        pltpu.sync_copy(dst_hbm.at[c, s], didx)

        h = CH // 4
        sems = None

        def edge_body(j, carry):
            cps = []
            for q, sm in enumerate((sem, sem2, sem3, sem4)):
                cps.append(pltpu.async_copy(
                    table_hbm.at[sidx.at[j, pl.ds(q * h, h)]],
                    rows.at[pl.ds(q * h, h)], sm))
            for cp in cps:
                cp.wait()
            pltpu.sync_copy(rows, acc.at[didx.at[j]], add=True)
            return carry
        lax.fori_loop(0, nch_c, edge_body, 0)

        plsc.subcore_barrier()
        _writeout(acc, out_hbm, c, s, n, tile_rows)

    return k


def _t1_body(degp, feat, fs, dinv):
    deg = degp[0, :, 0:1] + degp[1, :, 0:1] + 1.0
    di = lax.rsqrt(jnp.maximum(deg, 1.0))
    dinv[...] = di
    fs[...] = feat[...] * di


def _t2_body(aggp, fs, dinv, w1, b1, w2p, out):
    di = dinv[...]
    x = (aggp[0] + aggp[1] + fs[...]) * di
    h = jnp.maximum(jnp.dot(x, w1[...], preferred_element_type=jnp.float32)
                    + b1[...], 0.0)
    out[...] = jnp.dot(h, w2p[...], preferred_element_type=jnp.float32) * di


def _t3_body(agg2p, hw2s, dinv, b2p, out):
    t = (agg2p[0] + agg2p[1] + hw2s[...]) * dinv[...] + b2p[...]
    lg = t[:, :out.shape[1]]
    m = jnp.max(lg, axis=1, keepdims=True)
    e = jnp.exp(lg - m)
    out[...] = e / jnp.sum(e, axis=1, keepdims=True)


def kernel(feat, view, W1, b1, W2, b2):
    n, f_in = feat.shape
    e = view.shape[1]
    hid = W1.shape[1]
    c_out = W2.shape[1]
    cp = 128

    nw = NC * NS
    cg = NS * CH
    e0 = int(round(e * CORE0_FRAC / cg)) * cg
    e0 = max(cg, min(e0, (e // cg) * cg))
    e1 = e - e0
    nch0 = e0 // cg
    nch1 = _ceil_div(e1, cg)
    nch_max = max(nch0, nch1)
    nacc = _ceil_div(n + 1, NS * CH) * NS * CH

    def part(vals, pad_val):
        a = vals[:e0].reshape(NS, nch0, CH)
        a = jnp.concatenate(
            [a, jnp.full((NS, nch_max - nch0, CH), pad_val, jnp.int32)], axis=1)
        b = jnp.concatenate(
            [vals[e0:], jnp.full((nch1 * cg - e1,), pad_val, jnp.int32)])
        b = b.reshape(NS, nch1, CH)
        b = jnp.concatenate(
            [b, jnp.full((NS, nch_max - nch1, CH), pad_val, jnp.int32)], axis=1)
        return jnp.stack([a, b])

    src_p = part(view[0], 0)
    dst_p = part(view[1], n)

    w2p = jnp.concatenate(
        [W2, jnp.zeros((hid, cp - c_out), jnp.float32)], axis=1)
    b1r = b1.reshape(1, hid)
    b2p = jnp.concatenate(
        [b2, jnp.zeros((cp - c_out,), jnp.float32)]).reshape(1, cp)

    degp = _make_sc_deg(n, nacc, nch_max, nch0, nch1)(dst_p)

    r = 1000
    grid = (n // r,)
    fs, dinv = pl.pallas_call(
        _t1_body,
        grid=grid,
        in_specs=[pl.BlockSpec((NC, r, RW), lambda i: (0, i, 0)),
                  pl.BlockSpec((r, f_in), lambda i: (i, 0))],
        out_specs=[pl.BlockSpec((r, f_in), lambda i: (i, 0)),
                   pl.BlockSpec((r, 1), lambda i: (i, 0))],
        out_shape=[jax.ShapeDtypeStruct((n, f_in), jnp.float32),
                   jax.ShapeDtypeStruct((n, 1), jnp.float32)],
    )(degp, feat)

    aggp = _make_sc_agg(n, f_in, nacc, nch_max, nch0, nch1)(fs, src_p, dst_p)

    hw2s = pl.pallas_call(
        _t2_body,
        grid=grid,
        in_specs=[pl.BlockSpec((NC, r, f_in), lambda i: (0, i, 0)),
                  pl.BlockSpec((r, f_in), lambda i: (i, 0)),
                  pl.BlockSpec((r, 1), lambda i: (i, 0)),
                  pl.BlockSpec((f_in, hid), lambda i: (0, 0)),
                  pl.BlockSpec((1, hid), lambda i: (0, 0)),
                  pl.BlockSpec((hid, cp), lambda i: (0, 0))],
        out_specs=pl.BlockSpec((r, cp), lambda i: (i, 0)),
        out_shape=jax.ShapeDtypeStruct((n, cp), jnp.float32),
    )(aggp, fs, dinv, W1, b1r, w2p)

    agg2p = _make_sc_agg(n, cp, nacc, nch_max, nch0, nch1)(hw2s, src_p, dst_p)

    prob = pl.pallas_call(
        _t3_body,
        grid=grid,
        in_specs=[pl.BlockSpec((NC, r, cp), lambda i: (0, i, 0)),
                  pl.BlockSpec((r, cp), lambda i: (i, 0)),
                  pl.BlockSpec((r, 1), lambda i: (i, 0)),
                  pl.BlockSpec((1, cp), lambda i: (0, 0))],
        out_specs=pl.BlockSpec((r, c_out), lambda i: (i, 0)),
        out_shape=jax.ShapeDtypeStruct((n, c_out), jnp.float32),
    )(agg2p, hw2s, dinv, b2p)

    return prob
